# Initial kernel scaffold; baseline (speedup 1.0000x reference)
import jax
import jax.numpy as jnp
from jax.experimental import pallas as pl


def kernel(x, lut):
    raise NotImplementedError("write your pallas kernel here")


# speedup vs baseline: 4.2136x; 4.2136x over previous
import functools
import math

import jax
import jax.numpy as jnp
from jax import lax
from jax.experimental import pallas as pl
from jax.experimental.pallas import tpu as pltpu
from jax.experimental.pallas import tpu_sc as plsc

D_MODEL_K = 128
SCALE = math.sqrt(float(D_MODEL_K))

_info = plsc.get_sparse_core_info()
_NC, _NS, _L = _info.num_cores, _info.num_subcores, _info.num_lanes
_NW = _NC * _NS

_B_TOTAL = 1024 * 200
_B_PER_W = _B_TOTAL // _NW
_CHUNK = 128
_N_CHUNKS = _B_PER_W // _CHUNK


def _make_emb_kernel():
  mesh = plsc.VectorSubcoreMesh(core_axis_name="c", subcore_axis_name="s")

  @functools.partial(
      pl.kernel,
      mesh=mesh,
      out_type=jax.ShapeDtypeStruct((_B_TOTAL, D_MODEL_K), jnp.float32),
      scratch_types=[
          pltpu.VMEM((_CHUNK,), jnp.int32),
          pltpu.VMEM((_CHUNK, D_MODEL_K), jnp.float32),
          pltpu.SemaphoreType.DMA,
      ],
  )
  def emb(table_hbm, idx_hbm, out_hbm, idx_v, rows_v, sem):
    wid = lax.axis_index("s") * _NC + lax.axis_index("c")
    base = wid * _B_PER_W

    def chunk_body(j, carry):
      off = base + j * _CHUNK
      pltpu.sync_copy(idx_hbm.at[pl.ds(off, _CHUNK)], idx_v)
      pltpu.async_copy(table_hbm.at[idx_v], rows_v, sem).wait()

      def row_body(r, c):
        for g in range(D_MODEL_K // _L):
          sl = pl.ds(g * _L, _L)
          rows_v[r, sl] = rows_v[r, sl] * SCALE
        return c

      lax.fori_loop(0, _CHUNK, row_body, 0)
      pltpu.sync_copy(rows_v, out_hbm.at[pl.ds(off, _CHUNK)])
      return carry

    lax.fori_loop(0, _N_CHUNKS, chunk_body, 0)

  return emb


_emb = _make_emb_kernel()


@jax.jit
def kernel(x, lut):
  idx = x.reshape(-1).astype(jnp.int32)
  out = _emb(lut, idx)
  return out.reshape(x.shape[0], x.shape[1], D_MODEL_K)


# speedup vs baseline: 7.9096x; 1.8771x over previous
import functools
import math

import jax
import jax.numpy as jnp
from jax import lax
from jax.experimental import pallas as pl
from jax.experimental.pallas import tpu as pltpu
from jax.experimental.pallas import tpu_sc as plsc

D_MODEL_K = 128
SCALE = math.sqrt(float(D_MODEL_K))

_info = plsc.get_sparse_core_info()
_NC, _NS, _L = _info.num_cores, _info.num_subcores, _info.num_lanes
_NW = _NC * _NS

_B_TOTAL = 1024 * 200
_B_PER_W = _B_TOTAL // _NW
_CHUNK = 128
_N_CHUNKS = _B_PER_W // _CHUNK
_NBUF = 5
_ROW_UNROLL = 4


def _make_emb_kernel():
  mesh = plsc.VectorSubcoreMesh(core_axis_name="c", subcore_axis_name="s")

  scratch = [pltpu.VMEM((_B_PER_W,), jnp.int32)]
  scratch += [pltpu.VMEM((_CHUNK, D_MODEL_K), jnp.float32)] * _NBUF
  scratch += [pltpu.SemaphoreType.DMA] * (2 * _NBUF)

  @functools.partial(
      pl.kernel,
      mesh=mesh,
      out_type=jax.ShapeDtypeStruct((_B_TOTAL, D_MODEL_K), jnp.float32),
      scratch_types=scratch,
  )
  def emb(table_hbm, idx_hbm, out_hbm, idx_all, *bufs_and_sems):
    rows = bufs_and_sems[:_NBUF]
    gsem = bufs_and_sems[_NBUF:2 * _NBUF]
    ssem = bufs_and_sems[2 * _NBUF:]

    wid = lax.axis_index("s") * _NC + lax.axis_index("c")
    base = wid * _B_PER_W

    pltpu.sync_copy(idx_hbm.at[pl.ds(base, _B_PER_W)], idx_all)

    def start_gather(b, l):
      pltpu.make_async_copy(
          table_hbm.at[idx_all.at[pl.ds(l * _CHUNK, _CHUNK)]], rows[b],
          gsem[b]).start()

    def start_scatter(b, l):
      pltpu.make_async_copy(
          rows[b], out_hbm.at[pl.ds(base + l * _CHUNK, _CHUNK)],
          ssem[b]).start()

    def wait_gather(b):
      pltpu.make_async_copy(table_hbm.at[pl.ds(0, _CHUNK)], rows[b],
                            gsem[b]).wait()

    def wait_scatter(b):
      pltpu.make_async_copy(rows[b], out_hbm.at[pl.ds(0, _CHUNK)],
                            ssem[b]).wait()

    for b in range(_NBUF - 1):
      start_gather(b, b)

    def group_body(g, carry):
      for b in range(_NBUF):
        l = g + b
        bp = (b + _NBUF - 1) % _NBUF

        @pl.when(l + _NBUF - 1 < _N_CHUNKS)
        def _():
          @pl.when(l >= 1)
          def _():
            wait_scatter(bp)
          start_gather(bp, l + _NBUF - 1)

        wait_gather(b)

        def row_body(r4, c):
          for u in range(_ROW_UNROLL):
            r = r4 * _ROW_UNROLL + u
            for gcol in range(D_MODEL_K // _L):
              sl = pl.ds(gcol * _L, _L)
              rows[b][r, sl] = rows[b][r, sl] * SCALE
          return c

        lax.fori_loop(0, _CHUNK // _ROW_UNROLL, row_body, 0)
        start_scatter(b, l)
      return carry

    lax.fori_loop(0, _N_CHUNKS // _NBUF, lambda i, c: group_body(i * _NBUF, c),
                  0)

    for b in range(_NBUF):
      wait_scatter(b)

  return emb


_emb = _make_emb_kernel()


@jax.jit
def kernel(x, lut):
  idx = x.reshape(-1).astype(jnp.int32)
  out = _emb(lut, idx)
  return out.reshape(x.shape[0], x.shape[1], D_MODEL_K)
